# baseline (speedup 1.0000x reference)
import functools

import jax
import jax.numpy as jnp
from jax import lax
from jax.experimental import pallas as pl
from jax.experimental.pallas import tpu as pltpu
from jax.experimental.pallas import tpu_sc as plsc

_N = 10000
_E = 160000
_D = 256
_H = 512

_NS = 16
_C = 128
_NCHUNK = _E // _C
_NH = _N // 2
_TRASH = _NH
_RPT = 312
_MAXC = 79
_TRI = 27


def _sc_aggregate(h_v, src_e, dst, zer):
    mesh = plsc.VectorSubcoreMesh(core_axis_name="c", subcore_axis_name="s")

    @functools.partial(
        pl.kernel,
        out_type=jax.ShapeDtypeStruct((2, _N, 128), jnp.float32),
        mesh=mesh,
        scratch_types=[
            pltpu.VMEM((_MAXC * _C,), jnp.int32),
            pltpu.VMEM((_C,), jnp.int32),
            pltpu.VMEM((_C,), jnp.int32),
            pltpu.VMEM((_C,), jnp.int32),
            pltpu.VMEM((_C, 128), jnp.float32),
            pltpu.VMEM((_C, 128), jnp.float32),
            pltpu.VMEM((_C, 128), jnp.float32),
            pltpu.VMEM_SHARED((_NH + 8, 128), jnp.float32),
            pltpu.SemaphoreType.DMA,
            pltpu.SemaphoreType.DMA,
            pltpu.SemaphoreType.DMA,
            pltpu.SemaphoreType.DMA,
            pltpu.SemaphoreType.DMA,
            pltpu.SemaphoreType.DMA,
            pltpu.SemaphoreType.DMA,
            pltpu.SemaphoreType.DMA,
            pltpu.SemaphoreType.DMA,
            pltpu.SemaphoreType.DMA,
        ],
    )
    def k(h_hbm, src_hbm, dst_hbm, zer_hbm, out_hbm,
          src_all, di0, di1, di2, b0, b1_, b2_,
          acc, sem_i, sg0, sg1, sg2, ss0, ss1, ss2, sd0, sd1, sd2):
        cid = lax.axis_index("c")
        sid = lax.axis_index("s")
        nbase = cid * _NH
        nct = jnp.where(sid < 2, _MAXC, _MAXC - 1)
        bufs = (b0, b1_, b2_)
        dis = (di0, di1, di2)
        sgs = (sg0, sg1, sg2)
        sss = (ss0, ss1, ss2)
        sds = (sd0, sd1, sd2)

        def fire(m, carry):
            off = pl.multiple_of((sid + m * _NS) * _C, 128)
            v = pl.multiple_of(m * _C, 8)
            pltpu.async_copy(src_hbm.at[pl.ds(off, _C)],
                             src_all.at[pl.ds(v, _C)], sem_i)
            return carry

        lax.fori_loop(0, nct, fire, 0)

        def draini(m, carry):
            v = pl.multiple_of(m * _C, 8)
            pltpu.make_async_copy(src_hbm.at[pl.ds(0, _C)],
                                  src_all.at[pl.ds(v, _C)], sem_i).wait()
            return carry

        lax.fori_loop(0, nct, draini, 0)

        def dbl(i, carry):
            v = src_all[pl.ds(i * 16, 16)]
            src_all[pl.ds(i * 16, 16)] = v + v
            return carry

        lax.fori_loop(0, nct * (_C // 16), dbl, 0)

        r0 = sid * _RPT

        def start_dload(d_ref, m, sem):
            off = pl.multiple_of((sid + m * _NS) * _C, 128)
            pltpu.async_copy(dst_hbm.at[pl.ds(off, _C)], d_ref, sem)

        def wait_dload(d_ref, sem):
            pltpu.make_async_copy(dst_hbm.at[pl.ds(0, _C)], d_ref, sem).wait()

        def localize(d_ref):
            def go(i, carry):
                d = d_ref[pl.ds(i * 16, 16)] - nbase
                ok = (d >= 0) & (d < _NH)
                d_ref[pl.ds(i * 16, 16)] = jnp.where(ok, d, _TRASH)
                return carry

            lax.fori_loop(0, _C // 16, go, 0)

        def start_gather(buf, m, sem):
            off = pl.multiple_of(m * _C, 8)
            pltpu.async_copy(h_hbm.at[src_all.at[pl.ds(off, _C)]], buf, sem)

        def wait_gather(buf, sem):
            pltpu.make_async_copy(h_hbm.at[src_all.at[pl.ds(0, _C)]],
                                  buf, sem).wait()

        def start_scatter(buf, idx_ref, sem):
            pltpu.async_copy(buf, acc.at[idx_ref], sem, add=True)

        def wait_scatter(buf, idx_ref, sem):
            pltpu.make_async_copy(buf, acc.at[idx_ref], sem).wait()

        for p in range(2):
            if p == 1:
                def bump(i, carry):
                    v = src_all[pl.ds(i * 16, 16)]
                    src_all[pl.ds(i * 16, 16)] = v + 1
                    return carry

                lax.fori_loop(0, nct * (_C // 16), bump, 0)

            pltpu.sync_copy(zer_hbm, acc.at[pl.ds(r0, _RPT)])

            @pl.when(sid == _NS - 1)
            def _():
                pltpu.sync_copy(zer_hbm.at[pl.ds(0, 16)],
                                acc.at[pl.ds(_NS * _RPT, 16)])

            plsc.subcore_barrier()

            start_dload(di0, 0, sd0)
            start_gather(b0, 0, sg0)

            def pipe(j, carry):
                for t in range(3):
                    m = 3 * j + t
                    r = t
                    r1 = (t + 1) % 3

                    @pl.when(m < nct)
                    def _():
                        wait_gather(bufs[r], sgs[r])
                        wait_dload(dis[r], sds[r])
                        localize(dis[r])
                        start_scatter(bufs[r], dis[r], sss[r])

                    @pl.when((m >= 2) & (m - 2 < nct))
                    def _():
                        wait_scatter(bufs[r1], dis[r1], sss[r1])

                    @pl.when(m + 1 < nct)
                    def _():
                        start_dload(dis[r1], m + 1, sds[r1])
                        start_gather(bufs[r1], m + 1, sgs[r1])

                return carry

            lax.fori_loop(0, _TRI, pipe, 0)
            plsc.subcore_barrier()

            pltpu.sync_copy(acc.at[pl.ds(r0, _RPT)],
                            out_hbm.at[p, pl.ds(nbase + r0, _RPT)])

            @pl.when(sid == _NS - 1)
            def _():
                t0 = _NS * _RPT
                pltpu.sync_copy(acc.at[pl.ds(t0, 8)],
                                out_hbm.at[p, pl.ds(nbase + t0, 8)])

            plsc.subcore_barrier()

    return k(h_v, src_e, dst, zer)


_R = 1000


def _mlp_body(eps_ref, hr, alo, ahi, w1lo, w1hi, b1r, w2r, b2r,
              g1r, bt1r, g2r, bt2r, out_ref,
              z1_s, y_s, s1, q1, s2, q2, c1, c2):
    p = pl.program_id(0)
    i = pl.program_id(1)

    @pl.when(p == 0)
    def _():
        e1 = 1.0 + eps_ref[0, 0]
        hv = hr[...] * e1
        zlo = hv[:, :128] + alo[0]
        zhi = hv[:, 128:] + ahi[0]
        acc = jnp.dot(zlo.astype(jnp.bfloat16), w1lo[...].astype(jnp.bfloat16),
                      preferred_element_type=jnp.float32)
        acc = acc + jnp.dot(zhi.astype(jnp.bfloat16),
                            w1hi[...].astype(jnp.bfloat16),
                            preferred_element_type=jnp.float32)
        acc = acc + b1r[...]
        z1_s[pl.ds(i * _R, _R), :] = acc
        ps = jnp.sum(acc, axis=0, keepdims=True)
        pq = jnp.sum(acc * acc, axis=0, keepdims=True)

        @pl.when(i == 0)
        def _():
            s1[...] = ps
            q1[...] = pq

        @pl.when(i != 0)
        def _():
            s1[...] += ps
            q1[...] += pq

    @pl.when(p == 1)
    def _():
        @pl.when(i == 0)
        def _():
            mean = s1[...] / _N
            var = q1[...] / _N - mean * mean
            sc = g1r[...] * lax.rsqrt(var + 1e-5)
            c1[0:1, :] = sc
            c1[1:2, :] = bt1r[...] - mean * sc

        a = jnp.maximum(z1_s[pl.ds(i * _R, _R), :] * c1[0:1, :] + c1[1:2, :],
                        0.0)
        y = jnp.dot(a.astype(jnp.bfloat16), w2r[...].astype(jnp.bfloat16),
                    preferred_element_type=jnp.float32)
        y = y + b2r[...] + hr[...]
        y_s[pl.ds(i * _R, _R), :] = y
        ps = jnp.sum(y, axis=0, keepdims=True)
        pq = jnp.sum(y * y, axis=0, keepdims=True)

        @pl.when(i == 0)
        def _():
            s2[...] = ps
            q2[...] = pq

        @pl.when(i != 0)
        def _():
            s2[...] += ps
            q2[...] += pq

    @pl.when(p == 2)
    def _():
        @pl.when(i == 0)
        def _():
            mean = s2[...] / _N
            var = q2[...] / _N - mean * mean
            sc = g2r[...] * lax.rsqrt(var + 1e-5)
            c2[0:1, :] = sc
            c2[1:2, :] = bt2r[...] - mean * sc

        out_ref[...] = jnp.maximum(
            y_s[pl.ds(i * _R, _R), :] * c2[0:1, :] + c2[1:2, :], 0.0)


def _mlp_fused(h, agg2, W1, b1r, W2, b2r, g1r, bt1r, g2r, bt2r, eps11):
    grid = (3, _N // _R)
    fix = lambda bi: (lambda p, i: bi)
    return pl.pallas_call(
        _mlp_body,
        grid=grid,
        in_specs=[
            pl.BlockSpec((1, 1), fix((0, 0))),
            pl.BlockSpec((_R, _D), lambda p, i: (jnp.where(p < 2, i, 0), 0)),
            pl.BlockSpec((1, _R, 128),
                         lambda p, i: (0, jnp.where(p == 0, i, 0), 0)),
            pl.BlockSpec((1, _R, 128),
                         lambda p, i: (1, jnp.where(p == 0, i, 0), 0)),
            pl.BlockSpec((128, _H), fix((0, 0))),
            pl.BlockSpec((128, _H), fix((1, 0))),
            pl.BlockSpec((1, _H), fix((0, 0))),
            pl.BlockSpec((_H, _D), fix((0, 0))),
            pl.BlockSpec((1, _D), fix((0, 0))),
            pl.BlockSpec((1, _H), fix((0, 0))),
            pl.BlockSpec((1, _H), fix((0, 0))),
            pl.BlockSpec((1, _D), fix((0, 0))),
            pl.BlockSpec((1, _D), fix((0, 0))),
        ],
        out_specs=pl.BlockSpec((_R, _D),
                               lambda p, i: (jnp.where(p == 2, i, 0), 0)),
        out_shape=jax.ShapeDtypeStruct((_N, _D), jnp.float32),
        scratch_shapes=[
            pltpu.VMEM((_N, _H), jnp.float32),
            pltpu.VMEM((_N, _D), jnp.float32),
            pltpu.VMEM((1, _H), jnp.float32),
            pltpu.VMEM((1, _H), jnp.float32),
            pltpu.VMEM((1, _D), jnp.float32),
            pltpu.VMEM((1, _D), jnp.float32),
            pltpu.VMEM((2, _H), jnp.float32),
            pltpu.VMEM((2, _D), jnp.float32),
        ],
    )(eps11, h, agg2, agg2, W1, W1, b1r, W2, b2r, g1r, bt1r, g2r, bt2r)


def kernel(h, edge_index, W1, b1, gamma1, beta1, W2, b2, eps, gamma2, beta2):
    src = edge_index[0].astype(jnp.int32)
    dst = edge_index[1].astype(jnp.int32)
    h_v = h.reshape(2 * _N, 128)

    zer = jnp.zeros((_RPT, 128), jnp.float32)
    agg2 = _sc_aggregate(h_v, src, dst, zer)

    eps11 = (eps.astype(jnp.float32)).reshape(1, 1)
    return _mlp_fused(h, agg2, W1, b1.reshape(1, _H), W2, b2.reshape(1, _D),
                      gamma1.reshape(1, _H), beta1.reshape(1, _H),
                      gamma2.reshape(1, _D), beta2.reshape(1, _D), eps11)


# speedup vs baseline: 1.0101x; 1.0101x over previous
import functools

import jax
import jax.numpy as jnp
from jax import lax
from jax.experimental import pallas as pl
from jax.experimental.pallas import tpu as pltpu
from jax.experimental.pallas import tpu_sc as plsc

_N = 10000
_E = 160000
_D = 256
_H = 512

_NS = 16
_C = 128
_NCHUNK = _E // _C
_NH = _N // 2
_TRASH = _NH
_RPT = 312
_MAXC = 79
_TRI = 21


def _sc_aggregate(h_v, src2e, dst, zer):
    mesh = plsc.VectorSubcoreMesh(core_axis_name="c", subcore_axis_name="s")

    @functools.partial(
        pl.kernel,
        out_type=jax.ShapeDtypeStruct((2, _N, 128), jnp.float32),
        mesh=mesh,
        scratch_types=[
            pltpu.VMEM((_MAXC * _C,), jnp.int32),
            pltpu.VMEM((_C,), jnp.int32),
            pltpu.VMEM((_C,), jnp.int32),
            pltpu.VMEM((_C,), jnp.int32),
            pltpu.VMEM((_C,), jnp.int32),
            pltpu.VMEM((_C, 128), jnp.float32),
            pltpu.VMEM((_C, 128), jnp.float32),
            pltpu.VMEM((_C, 128), jnp.float32),
            pltpu.VMEM((_C, 128), jnp.float32),
            pltpu.VMEM_SHARED((_NH + 8, 128), jnp.float32),
            pltpu.SemaphoreType.DMA,
            pltpu.SemaphoreType.DMA,
            pltpu.SemaphoreType.DMA,
            pltpu.SemaphoreType.DMA,
            pltpu.SemaphoreType.DMA,
            pltpu.SemaphoreType.DMA,
            pltpu.SemaphoreType.DMA,
            pltpu.SemaphoreType.DMA,
            pltpu.SemaphoreType.DMA,
            pltpu.SemaphoreType.DMA,
            pltpu.SemaphoreType.DMA,
            pltpu.SemaphoreType.DMA,
            pltpu.SemaphoreType.DMA,
        ],
    )
    def k(h_hbm, src_hbm, dst_hbm, zer_hbm, out_hbm,
          src_all, di0, di1, di2, di3, b0, b1_, b2_, b3_,
          acc, sem_i, sg0, sg1, sg2, sg3, ss0, ss1, ss2, ss3,
          sd0, sd1, sd2, sd3):
        cid = lax.axis_index("c")
        sid = lax.axis_index("s")
        nbase = cid * _NH
        nct = jnp.where(sid < 2, _MAXC, _MAXC - 1)
        bufs = (b0, b1_, b2_, b3_)
        dis = (di0, di1, di2, di3)
        sgs = (sg0, sg1, sg2, sg3)
        sss = (ss0, ss1, ss2, ss3)
        sds = (sd0, sd1, sd2, sd3)

        def fire(m, carry):
            off = pl.multiple_of((sid + m * _NS) * _C, 128)
            v = pl.multiple_of(m * _C, 8)
            pltpu.async_copy(src_hbm.at[pl.ds(off, _C)],
                             src_all.at[pl.ds(v, _C)], sem_i)
            return carry

        lax.fori_loop(0, nct, fire, 0)

        def draini(m, carry):
            v = pl.multiple_of(m * _C, 8)
            pltpu.make_async_copy(src_hbm.at[pl.ds(0, _C)],
                                  src_all.at[pl.ds(v, _C)], sem_i).wait()
            return carry

        lax.fori_loop(0, nct, draini, 0)

        r0 = sid * _RPT

        def start_dload(d_ref, m, sem):
            off = pl.multiple_of((sid + m * _NS) * _C, 128)
            pltpu.async_copy(dst_hbm.at[pl.ds(off, _C)], d_ref, sem)

        def wait_dload(d_ref, sem):
            pltpu.make_async_copy(dst_hbm.at[pl.ds(0, _C)], d_ref, sem).wait()

        def localize(d_ref):
            def go(i, carry):
                d = d_ref[pl.ds(i * 16, 16)] - nbase
                ok = (d >= 0) & (d < _NH)
                d_ref[pl.ds(i * 16, 16)] = jnp.where(ok, d, _TRASH)
                return carry

            lax.fori_loop(0, _C // 16, go, 0)

        def start_gather(buf, m, sem):
            off = pl.multiple_of(m * _C, 8)
            pltpu.async_copy(h_hbm.at[src_all.at[pl.ds(off, _C)]], buf, sem)

        def wait_gather(buf, sem):
            pltpu.make_async_copy(h_hbm.at[src_all.at[pl.ds(0, _C)]],
                                  buf, sem).wait()

        def start_scatter(buf, idx_ref, sem):
            pltpu.async_copy(buf, acc.at[idx_ref], sem, add=True)

        def wait_scatter(buf, idx_ref, sem):
            pltpu.make_async_copy(buf, acc.at[idx_ref], sem).wait()

        for p in range(2):
            if p == 1:
                def bump(i, carry):
                    v = src_all[pl.ds(i * 16, 16)]
                    src_all[pl.ds(i * 16, 16)] = v + 1
                    return carry

                lax.fori_loop(0, nct * (_C // 16), bump, 0)

            pltpu.sync_copy(zer_hbm, acc.at[pl.ds(r0, _RPT)])

            @pl.when(sid == _NS - 1)
            def _():
                pltpu.sync_copy(zer_hbm.at[pl.ds(0, 16)],
                                acc.at[pl.ds(_NS * _RPT, 16)])

            plsc.subcore_barrier()

            start_dload(di0, 0, sd0)
            start_gather(b0, 0, sg0)

            def pipe(j, carry):
                for t in range(4):
                    m = 4 * j + t
                    r = t
                    r1 = (t + 1) % 4

                    @pl.when(m < nct)
                    def _():
                        wait_gather(bufs[r], sgs[r])
                        wait_dload(dis[r], sds[r])
                        localize(dis[r])
                        start_scatter(bufs[r], dis[r], sss[r])

                    @pl.when((m >= 3) & (m - 3 < nct))
                    def _():
                        wait_scatter(bufs[r1], dis[r1], sss[r1])

                    @pl.when(m + 1 < nct)
                    def _():
                        start_dload(dis[r1], m + 1, sds[r1])
                        start_gather(bufs[r1], m + 1, sgs[r1])

                return carry

            lax.fori_loop(0, _TRI, pipe, 0)
            plsc.subcore_barrier()

            pltpu.sync_copy(acc.at[pl.ds(r0, _RPT)],
                            out_hbm.at[p, pl.ds(nbase + r0, _RPT)])

            @pl.when(sid == _NS - 1)
            def _():
                t0 = _NS * _RPT
                pltpu.sync_copy(acc.at[pl.ds(t0, 8)],
                                out_hbm.at[p, pl.ds(nbase + t0, 8)])

            plsc.subcore_barrier()

    return k(h_v, src2e, dst, zer)


_R = 1000


def _mlp_body(eps_ref, hr, alo, ahi, w1lo, w1hi, b1r, w2r, b2r,
              g1r, bt1r, g2r, bt2r, out_ref,
              z1_s, y_s, s1, q1, s2, q2, c1, c2):
    p = pl.program_id(0)
    i = pl.program_id(1)

    @pl.when(p == 0)
    def _():
        e1 = 1.0 + eps_ref[0, 0]
        hv = hr[...] * e1
        zlo = hv[:, :128] + alo[0]
        zhi = hv[:, 128:] + ahi[0]
        acc = jnp.dot(zlo, w1lo[...], preferred_element_type=jnp.float32)
        acc = acc + jnp.dot(zhi, w1hi[...], preferred_element_type=jnp.float32)
        acc = acc + b1r[...]
        z1_s[pl.ds(i * _R, _R), :] = acc
        ps = jnp.sum(acc, axis=0, keepdims=True)
        pq = jnp.sum(acc * acc, axis=0, keepdims=True)

        @pl.when(i == 0)
        def _():
            s1[...] = ps
            q1[...] = pq

        @pl.when(i != 0)
        def _():
            s1[...] += ps
            q1[...] += pq

    @pl.when(p == 1)
    def _():
        @pl.when(i == 0)
        def _():
            mean = s1[...] / _N
            var = q1[...] / _N - mean * mean
            sc = g1r[...] * lax.rsqrt(var + 1e-5)
            c1[0:1, :] = sc
            c1[1:2, :] = bt1r[...] - mean * sc

        a = jnp.maximum(z1_s[pl.ds(i * _R, _R), :] * c1[0:1, :] + c1[1:2, :],
                        0.0)
        y = jnp.dot(a, w2r[...], preferred_element_type=jnp.float32)
        y = y + b2r[...] + hr[...]
        y_s[pl.ds(i * _R, _R), :] = y
        ps = jnp.sum(y, axis=0, keepdims=True)
        pq = jnp.sum(y * y, axis=0, keepdims=True)

        @pl.when(i == 0)
        def _():
            s2[...] = ps
            q2[...] = pq

        @pl.when(i != 0)
        def _():
            s2[...] += ps
            q2[...] += pq

    @pl.when(p == 2)
    def _():
        @pl.when(i == 0)
        def _():
            mean = s2[...] / _N
            var = q2[...] / _N - mean * mean
            sc = g2r[...] * lax.rsqrt(var + 1e-5)
            c2[0:1, :] = sc
            c2[1:2, :] = bt2r[...] - mean * sc

        out_ref[...] = jnp.maximum(
            y_s[pl.ds(i * _R, _R), :] * c2[0:1, :] + c2[1:2, :], 0.0)


def _mlp_fused(h, agg2, W1, b1r, W2, b2r, g1r, bt1r, g2r, bt2r, eps11):
    grid = (3, _N // _R)
    fix = lambda bi: (lambda p, i: bi)
    return pl.pallas_call(
        _mlp_body,
        grid=grid,
        in_specs=[
            pl.BlockSpec((1, 1), fix((0, 0))),
            pl.BlockSpec((_R, _D), lambda p, i: (jnp.where(p < 2, i, 0), 0)),
            pl.BlockSpec((1, _R, 128),
                         lambda p, i: (0, jnp.where(p == 0, i, 0), 0)),
            pl.BlockSpec((1, _R, 128),
                         lambda p, i: (1, jnp.where(p == 0, i, 0), 0)),
            pl.BlockSpec((128, _H), fix((0, 0))),
            pl.BlockSpec((128, _H), fix((1, 0))),
            pl.BlockSpec((1, _H), fix((0, 0))),
            pl.BlockSpec((_H, _D), fix((0, 0))),
            pl.BlockSpec((1, _D), fix((0, 0))),
            pl.BlockSpec((1, _H), fix((0, 0))),
            pl.BlockSpec((1, _H), fix((0, 0))),
            pl.BlockSpec((1, _D), fix((0, 0))),
            pl.BlockSpec((1, _D), fix((0, 0))),
        ],
        out_specs=pl.BlockSpec((_R, _D),
                               lambda p, i: (jnp.where(p == 2, i, 0), 0)),
        out_shape=jax.ShapeDtypeStruct((_N, _D), jnp.float32),
        scratch_shapes=[
            pltpu.VMEM((_N, _H), jnp.float32),
            pltpu.VMEM((_N, _D), jnp.float32),
            pltpu.VMEM((1, _H), jnp.float32),
            pltpu.VMEM((1, _H), jnp.float32),
            pltpu.VMEM((1, _D), jnp.float32),
            pltpu.VMEM((1, _D), jnp.float32),
            pltpu.VMEM((2, _H), jnp.float32),
            pltpu.VMEM((2, _D), jnp.float32),
        ],
    )(eps11, h, agg2, agg2, W1, W1, b1r, W2, b2r, g1r, bt1r, g2r, bt2r)


def kernel(h, edge_index, W1, b1, gamma1, beta1, W2, b2, eps, gamma2, beta2):
    src = edge_index[0].astype(jnp.int32)
    dst = edge_index[1].astype(jnp.int32)
    src2e = src * 2
    h_v = h.reshape(2 * _N, 128)

    zer = jnp.zeros((_RPT, 128), jnp.float32)
    agg2 = _sc_aggregate(h_v, src2e, dst, zer)

    eps11 = (eps.astype(jnp.float32)).reshape(1, 1)
    return _mlp_fused(h, agg2, W1, b1.reshape(1, _H), W2, b2.reshape(1, _D),
                      gamma1.reshape(1, _H), beta1.reshape(1, _H),
                      gamma2.reshape(1, _D), beta2.reshape(1, _D), eps11)
